# baseline (speedup 1.0000x reference)
import jax
import jax.numpy as jnp
from jax.experimental import pallas as pl
from jax.experimental.pallas import tpu as pltpu
from jax.experimental.pallas import tpu_sc as plsc

_RANK = 16
_GATHER_WINDOW = 128
_TOKEN_BLOCK = 512


def _sc_gather(bpe4, root_emb, affix128, tok4, rid, aid):
    n4 = tok4.shape[0]
    n = rid.shape[0]
    w = _GATHER_WINDOW
    mesh = plsc.VectorSubcoreMesh(core_axis_name="c", subcore_axis_name="s")
    out_type = (
        jax.ShapeDtypeStruct((n4, 128), jnp.float32),
        jax.ShapeDtypeStruct((n, 128), jnp.float32),
        jax.ShapeDtypeStruct((n, 128), jnp.float32),
    )

    @pl.kernel(out_type=out_type, mesh=mesh)
    def gather_kernel(bpe_hbm, root_hbm, affix_hbm, tok_hbm, rid_hbm, aid_hbm,
                      bpe_out, root_out, affix_out):
        def bpe_body(tok_v, bpe_v):
            pltpu.sync_copy(bpe_hbm.at[tok_v.at[0]], bpe_v)

        pltpu.emit_pipeline(
            bpe_body,
            grid=(n4 // w,),
            in_specs=[pl.BlockSpec((1, w), lambda i: (0, i))],
            out_specs=[pl.BlockSpec((w, 128), lambda i: (i, 0))],
            core_axis_name=("c", "s"),
            dimension_semantics=(pltpu.PARALLEL,),
        )(tok_hbm, bpe_out)

        def small_body(rid_v, aid_v, root_v, affix_v):
            pltpu.sync_copy(root_hbm.at[rid_v.at[0]], root_v)
            pltpu.sync_copy(affix_hbm.at[aid_v.at[0]], affix_v)

        pltpu.emit_pipeline(
            small_body,
            grid=(n // w,),
            in_specs=[
                pl.BlockSpec((1, w), lambda i: (0, i)),
                pl.BlockSpec((1, w), lambda i: (0, i)),
            ],
            out_specs=[
                pl.BlockSpec((w, 128), lambda i: (i, 0)),
                pl.BlockSpec((w, 128), lambda i: (i, 0)),
            ],
            core_axis_name=("c", "s"),
            dimension_semantics=(pltpu.PARALLEL,),
        )(rid_hbm, aid_hbm, root_out, affix_out)

    return gather_kernel(bpe4, root_emb, affix128,
                         tok4.reshape(1, n4), rid.reshape(1, n),
                         aid.reshape(1, n))


def _morph_body(pat_ref, rid_ref, root_ref, affix_ref, bpe_ref,
                vu_ref, uu_ref, w1t_ref, b1_ref, w2t_ref, b2_ref, gb_ref,
                out_ref):
    num_pat = (vu_ref.shape[1] // _RANK) - 1

    bf = jnp.bfloat16
    x = root_ref[...].astype(bf)
    a = jax.lax.dot_general(x, vu_ref[...].astype(bf),
                            (((1,), (0,)), ((), ())),
                            preferred_element_type=jnp.float32)
    pat = pat_ref[...]
    safe_pat = jnp.clip(pat, 0, num_pat - 1)
    col = jax.lax.broadcasted_iota(jnp.int32, a.shape, 1) // _RANK
    mask = (col == safe_pat) | (col == num_pat)
    a = jnp.where(mask, a, 0.0).astype(bf)
    morph = jax.lax.dot_general(a, uu_ref[...].astype(bf),
                                (((1,), (0,)), ((), ())),
                                preferred_element_type=jnp.float32)
    affix_dim = w1t_ref.shape[0] - uu_ref.shape[1]
    mf = jnp.concatenate([morph.astype(bf), affix_ref[:, :affix_dim].astype(bf)],
                         axis=1)
    h = jax.lax.dot_general(mf, w1t_ref[...].astype(bf),
                            (((1,), (0,)), ((), ())),
                            preferred_element_type=jnp.float32) + b1_ref[...]
    h = 0.5 * h * (1.0 + jax.lax.erf(h * 0.7071067811865476))
    o = jax.lax.dot_general(h.astype(bf), w2t_ref[...].astype(bf),
                            (((1,), (0,)), ((), ())),
                            preferred_element_type=jnp.float32) + b2_ref[...]
    gate = jax.nn.sigmoid(gb_ref[0, 0])
    bpe = bpe_ref[...]
    has_morph = (rid_ref[...] >= 0) & (pat >= 0)
    out_ref[...] = jnp.where(has_morph, gate * o + (1.0 - gate) * bpe, bpe)


def _tc_compute(pat, rid, root_vecs, affix_vecs, bpe_emb,
                vu, uu, w1t, b1, w2t, b2, gate_bias):
    n, dim = bpe_emb.shape
    t = _TOKEN_BLOCK
    grid = (n // t,)

    def tok_spec(shape):
        nd = len(shape)
        return pl.BlockSpec((t,) + tuple(shape[1:]),
                            lambda i, nd=nd: (i,) + (0,) * (nd - 1))

    def full_spec(arr):
        nd = arr.ndim
        return pl.BlockSpec(arr.shape, lambda i, nd=nd: (0,) * nd)

    return pl.pallas_call(
        _morph_body,
        grid=grid,
        in_specs=[
            tok_spec(pat.shape),
            tok_spec(rid.shape),
            tok_spec(root_vecs.shape),
            tok_spec(affix_vecs.shape),
            tok_spec(bpe_emb.shape),
            full_spec(vu),
            full_spec(uu),
            full_spec(w1t),
            full_spec(b1),
            full_spec(w2t),
            full_spec(b2),
            full_spec(gate_bias),
        ],
        out_specs=tok_spec((n, dim)),
        out_shape=jax.ShapeDtypeStruct((n, dim), jnp.float32),
    )(pat, rid, root_vecs, affix_vecs, bpe_emb, vu, uu, w1t, b1, w2t, b2,
      gate_bias)


def kernel(token_ids, root_ids, pattern_ids, affix_ids, root_emb, transform_U,
           transform_V, shared_U, shared_V, affix_emb, W1, b1, W2, b2,
           bpe_table, gate_bias):
    b, s = token_ids.shape
    n = b * s
    dim = bpe_table.shape[1]
    num_pat = transform_U.shape[0]
    root_dim = root_emb.shape[1]

    tok = jnp.clip(token_ids.reshape(n).astype(jnp.int32), 0, bpe_table.shape[0] - 1)
    rid_raw = root_ids.reshape(n, 1).astype(jnp.int32)
    pat_raw = pattern_ids.reshape(n, 1).astype(jnp.int32)
    rid_safe = jnp.clip(rid_raw[:, 0], 0, root_emb.shape[0] - 1)
    aid_safe = jnp.clip(affix_ids.reshape(n).astype(jnp.int32), 0, affix_emb.shape[0] - 1)

    splits = dim // 128
    bpe4 = bpe_table.reshape(bpe_table.shape[0] * splits, 128)
    tok4 = (tok[:, None] * splits + jnp.arange(splits, dtype=jnp.int32)[None, :]).reshape(n * splits)
    affix128 = jnp.pad(affix_emb, ((0, 0), (0, 128 - affix_emb.shape[1])))

    bpe4_emb, root_vecs, affix_vecs = _sc_gather(
        bpe4, root_emb, affix128, tok4, rid_safe, aid_safe)
    bpe_emb = bpe4_emb.reshape(n, dim)

    vu = jnp.concatenate(
        [jnp.transpose(transform_V, (2, 0, 1)).reshape(root_dim, num_pat * _RANK),
         shared_V.T], axis=1)
    uu = jnp.concatenate(
        [jnp.transpose(transform_U, (0, 2, 1)).reshape(num_pat * _RANK, root_dim),
         shared_U.T], axis=0)

    out = _tc_compute(pat_raw, rid_raw, root_vecs, affix_vecs, bpe_emb,
                      vu, uu, W1.T, b1.reshape(1, dim), W2.T, b2.reshape(1, dim),
                      gate_bias.reshape(1, 1))
    return out.reshape(b, s, dim)


# speedup vs baseline: 2.1922x; 2.1922x over previous
import jax
import jax.numpy as jnp
from jax.experimental import pallas as pl
from jax.experimental.pallas import tpu as pltpu
from jax.experimental.pallas import tpu_sc as plsc

_RANK = 16
_GATHER_WINDOW = 128
_TOKEN_BLOCK = 512


def _sc_gather(bpe_table, root_emb, affix128, tok, rid, aid):
    n = rid.shape[0]
    w = _GATHER_WINDOW
    dim = bpe_table.shape[1]
    mesh = plsc.VectorSubcoreMesh(core_axis_name="c", subcore_axis_name="s")
    out_type = (
        jax.ShapeDtypeStruct((n, dim), jnp.float32),
        jax.ShapeDtypeStruct((n, 128), jnp.float32),
        jax.ShapeDtypeStruct((n, 128), jnp.float32),
    )

    num_units = mesh.num_cores * mesh.num_subcores
    blocks_per_unit = n // (w * num_units)
    half = w // 2

    @pl.kernel(out_type=out_type, mesh=mesh,
               scratch_types=[
                   pltpu.VMEM((1, w), jnp.int32),
                   pltpu.VMEM((1, w), jnp.int32),
                   pltpu.VMEM((1, w), jnp.int32),
                   pltpu.VMEM((half, dim), jnp.float32),
                   pltpu.VMEM((half, dim), jnp.float32),
                   pltpu.VMEM((w, 128), jnp.float32),
                   pltpu.VMEM((w, 128), jnp.float32),
                   pltpu.SemaphoreType.DMA,
                   pltpu.SemaphoreType.DMA,
                   pltpu.SemaphoreType.DMA,
                   pltpu.SemaphoreType.DMA,
                   pltpu.SemaphoreType.DMA,
               ])
    def gather_kernel(bpe_hbm, root_hbm, affix_hbm, tok_hbm, rid_hbm, aid_hbm,
                      bpe_out, root_out, affix_out,
                      idx_t, idx_r, idx_a, buf_a, buf_b, buf_root, buf_affix,
                      sem_i, sem_a, sem_b, sem_r, sem_x):
        unit = jax.lax.axis_index("c") * mesh.num_subcores + jax.lax.axis_index("s")

        @pl.loop(0, blocks_per_unit)
        def _(i):
            base = (unit * blocks_per_unit + i) * w
            ld_t = pltpu.async_copy(tok_hbm.at[0, pl.ds(base, w)], idx_t.at[0], sem_i)
            ld_r = pltpu.async_copy(rid_hbm.at[0, pl.ds(base, w)], idx_r.at[0], sem_i)
            ld_a = pltpu.async_copy(aid_hbm.at[0, pl.ds(base, w)], idx_a.at[0], sem_i)
            ld_t.wait()
            ld_r.wait()
            ld_a.wait()
            pltpu.sync_copy(bpe_hbm.at[idx_t.at[0, pl.ds(0, half)]], buf_a)
            wb_a = pltpu.async_copy(buf_a, bpe_out.at[pl.ds(base, half)], sem_a)
            pltpu.sync_copy(bpe_hbm.at[idx_t.at[0, pl.ds(half, half)]], buf_b)
            wb_b = pltpu.async_copy(buf_b, bpe_out.at[pl.ds(base + half, half)],
                                    sem_b)
            pltpu.sync_copy(root_hbm.at[idx_r.at[0]], buf_root)
            wb_r = pltpu.async_copy(buf_root, root_out.at[pl.ds(base, w)], sem_r)
            pltpu.sync_copy(affix_hbm.at[idx_a.at[0]], buf_affix)
            wb_x = pltpu.async_copy(buf_affix, affix_out.at[pl.ds(base, w)], sem_x)
            wb_a.wait()
            wb_b.wait()
            wb_r.wait()
            wb_x.wait()

    return gather_kernel(bpe_table, root_emb, affix128,
                         tok.reshape(1, n), rid.reshape(1, n),
                         aid.reshape(1, n))


def _morph_body(pat_ref, rid_ref, root_ref, affix_ref, bpe_ref,
                vu_ref, uu_ref, w1t_ref, b1_ref, w2t_ref, b2_ref, gb_ref,
                out_ref):
    num_pat = (vu_ref.shape[1] // _RANK) - 1

    bf = jnp.bfloat16
    x = root_ref[...].astype(bf)
    a = jax.lax.dot_general(x, vu_ref[...].astype(bf),
                            (((1,), (0,)), ((), ())),
                            preferred_element_type=jnp.float32)
    pat = pat_ref[...]
    safe_pat = jnp.clip(pat, 0, num_pat - 1)
    col = jax.lax.broadcasted_iota(jnp.int32, a.shape, 1) // _RANK
    mask = (col == safe_pat) | (col == num_pat)
    a = jnp.where(mask, a, 0.0).astype(bf)
    morph = jax.lax.dot_general(a, uu_ref[...].astype(bf),
                                (((1,), (0,)), ((), ())),
                                preferred_element_type=jnp.float32)
    affix_dim = w1t_ref.shape[0] - uu_ref.shape[1]
    mf = jnp.concatenate([morph.astype(bf), affix_ref[:, :affix_dim].astype(bf)],
                         axis=1)
    h = jax.lax.dot_general(mf, w1t_ref[...].astype(bf),
                            (((1,), (0,)), ((), ())),
                            preferred_element_type=jnp.float32) + b1_ref[...]
    h = 0.5 * h * (1.0 + jax.lax.erf(h * 0.7071067811865476))
    o = jax.lax.dot_general(h.astype(bf), w2t_ref[...].astype(bf),
                            (((1,), (0,)), ((), ())),
                            preferred_element_type=jnp.float32) + b2_ref[...]
    gate = jax.nn.sigmoid(gb_ref[0, 0])
    bpe = bpe_ref[...]
    has_morph = (rid_ref[...] >= 0) & (pat >= 0)
    out_ref[...] = jnp.where(has_morph, gate * o + (1.0 - gate) * bpe, bpe)


def _tc_compute(pat, rid, root_vecs, affix_vecs, bpe_emb,
                vu, uu, w1t, b1, w2t, b2, gate_bias):
    n, dim = bpe_emb.shape
    t = _TOKEN_BLOCK
    grid = (n // t,)

    def tok_spec(shape):
        nd = len(shape)
        return pl.BlockSpec((t,) + tuple(shape[1:]),
                            lambda i, nd=nd: (i,) + (0,) * (nd - 1))

    def full_spec(arr):
        nd = arr.ndim
        return pl.BlockSpec(arr.shape, lambda i, nd=nd: (0,) * nd)

    return pl.pallas_call(
        _morph_body,
        grid=grid,
        in_specs=[
            tok_spec(pat.shape),
            tok_spec(rid.shape),
            tok_spec(root_vecs.shape),
            tok_spec(affix_vecs.shape),
            tok_spec(bpe_emb.shape),
            full_spec(vu),
            full_spec(uu),
            full_spec(w1t),
            full_spec(b1),
            full_spec(w2t),
            full_spec(b2),
            full_spec(gate_bias),
        ],
        out_specs=tok_spec((n, dim)),
        out_shape=jax.ShapeDtypeStruct((n, dim), jnp.float32),
    )(pat, rid, root_vecs, affix_vecs, bpe_emb, vu, uu, w1t, b1, w2t, b2,
      gate_bias)


def kernel(token_ids, root_ids, pattern_ids, affix_ids, root_emb, transform_U,
           transform_V, shared_U, shared_V, affix_emb, W1, b1, W2, b2,
           bpe_table, gate_bias):
    b, s = token_ids.shape
    n = b * s
    dim = bpe_table.shape[1]
    num_pat = transform_U.shape[0]
    root_dim = root_emb.shape[1]

    tok = jnp.clip(token_ids.reshape(n).astype(jnp.int32), 0, bpe_table.shape[0] - 1)
    rid_raw = root_ids.reshape(n, 1).astype(jnp.int32)
    pat_raw = pattern_ids.reshape(n, 1).astype(jnp.int32)
    rid_safe = jnp.clip(rid_raw[:, 0], 0, root_emb.shape[0] - 1)
    aid_safe = jnp.clip(affix_ids.reshape(n).astype(jnp.int32), 0, affix_emb.shape[0] - 1)

    affix128 = jnp.pad(affix_emb, ((0, 0), (0, 128 - affix_emb.shape[1])))

    bpe_emb, root_vecs, affix_vecs = _sc_gather(
        bpe_table, root_emb, affix128, tok, rid_safe, aid_safe)

    vu = jnp.concatenate(
        [jnp.transpose(transform_V, (2, 0, 1)).reshape(root_dim, num_pat * _RANK),
         shared_V.T], axis=1)
    uu = jnp.concatenate(
        [jnp.transpose(transform_U, (0, 2, 1)).reshape(num_pat * _RANK, root_dim),
         shared_U.T], axis=0)

    out = _tc_compute(pat_raw, rid_raw, root_vecs, affix_vecs, bpe_emb,
                      vu, uu, W1.T, b1.reshape(1, dim), W2.T, b2.reshape(1, dim),
                      gate_bias.reshape(1, 1))
    return out.reshape(b, s, dim)


# speedup vs baseline: 2.4485x; 1.1169x over previous
import jax
import jax.numpy as jnp
from jax.experimental import pallas as pl
from jax.experimental.pallas import tpu as pltpu
from jax.experimental.pallas import tpu_sc as plsc

_RANK = 16
_GATHER_WINDOW = 128
_TOKEN_BLOCK = 1024


def _sc_gather(bpe_table, root_emb, affix128, tok, rid, aid):
    n = rid.shape[0]
    w = _GATHER_WINDOW
    dim = bpe_table.shape[1]
    mesh = plsc.VectorSubcoreMesh(core_axis_name="c", subcore_axis_name="s")
    out_type = (
        jax.ShapeDtypeStruct((n, dim), jnp.float32),
        jax.ShapeDtypeStruct((n, 128), jnp.float32),
        jax.ShapeDtypeStruct((n, 128), jnp.float32),
    )

    num_units = mesh.num_cores * mesh.num_subcores
    blocks_per_unit = n // (w * num_units)
    half = w // 2

    @pl.kernel(out_type=out_type, mesh=mesh,
               scratch_types=[
                   pltpu.VMEM((1, w), jnp.int32),
                   pltpu.VMEM((1, w), jnp.int32),
                   pltpu.VMEM((1, w), jnp.int32),
                   pltpu.VMEM((half, dim), jnp.float32),
                   pltpu.VMEM((half, dim), jnp.float32),
                   pltpu.VMEM((w, 128), jnp.float32),
                   pltpu.VMEM((w, 128), jnp.float32),
                   pltpu.SemaphoreType.DMA,
                   pltpu.SemaphoreType.DMA,
                   pltpu.SemaphoreType.DMA,
                   pltpu.SemaphoreType.DMA,
                   pltpu.SemaphoreType.DMA,
               ])
    def gather_kernel(bpe_hbm, root_hbm, affix_hbm, tok_hbm, rid_hbm, aid_hbm,
                      bpe_out, root_out, affix_out,
                      idx_t, idx_r, idx_a, buf_a, buf_b, buf_root, buf_affix,
                      sem_i, sem_a, sem_b, sem_r, sem_x):
        unit = jax.lax.axis_index("c") * mesh.num_subcores + jax.lax.axis_index("s")

        @pl.loop(0, blocks_per_unit)
        def _(i):
            base = (unit * blocks_per_unit + i) * w
            ld_t = pltpu.async_copy(tok_hbm.at[0, pl.ds(base, w)], idx_t.at[0], sem_i)
            ld_r = pltpu.async_copy(rid_hbm.at[0, pl.ds(base, w)], idx_r.at[0], sem_i)
            ld_a = pltpu.async_copy(aid_hbm.at[0, pl.ds(base, w)], idx_a.at[0], sem_i)
            ld_t.wait()
            ld_r.wait()
            ld_a.wait()
            pltpu.sync_copy(bpe_hbm.at[idx_t.at[0, pl.ds(0, half)]], buf_a)
            wb_a = pltpu.async_copy(buf_a, bpe_out.at[pl.ds(base, half)], sem_a)
            pltpu.sync_copy(bpe_hbm.at[idx_t.at[0, pl.ds(half, half)]], buf_b)
            wb_b = pltpu.async_copy(buf_b, bpe_out.at[pl.ds(base + half, half)],
                                    sem_b)
            pltpu.sync_copy(root_hbm.at[idx_r.at[0]], buf_root)
            wb_r = pltpu.async_copy(buf_root, root_out.at[pl.ds(base, w)], sem_r)
            pltpu.sync_copy(affix_hbm.at[idx_a.at[0]], buf_affix)
            wb_x = pltpu.async_copy(buf_affix, affix_out.at[pl.ds(base, w)], sem_x)
            wb_a.wait()
            wb_b.wait()
            wb_r.wait()
            wb_x.wait()

    return gather_kernel(bpe_table, root_emb, affix128,
                         tok.reshape(1, n), rid.reshape(1, n),
                         aid.reshape(1, n))


def _morph_body(pat_ref, rid_ref, root_ref, affix_ref, bpe_ref,
                vu_ref, uu_ref, w1t_ref, b1_ref, w2t_ref, b2_ref, gb_ref,
                out_ref):
    num_pat = (vu_ref.shape[1] // _RANK) - 1

    bf = jnp.bfloat16
    x = root_ref[...].astype(bf)
    a = jax.lax.dot_general(x, vu_ref[...].astype(bf),
                            (((1,), (0,)), ((), ())),
                            preferred_element_type=jnp.float32)
    pat = pat_ref[...]
    safe_pat = jnp.clip(pat, 0, num_pat - 1)
    col = jax.lax.broadcasted_iota(jnp.int32, a.shape, 1) // _RANK
    mask = (col == safe_pat) | (col == num_pat)
    a = jnp.where(mask, a, 0.0).astype(bf)
    morph = jax.lax.dot_general(a, uu_ref[...].astype(bf),
                                (((1,), (0,)), ((), ())),
                                preferred_element_type=jnp.float32)
    affix_dim = w1t_ref.shape[0] - uu_ref.shape[1]
    mf = jnp.concatenate([morph.astype(bf), affix_ref[:, :affix_dim].astype(bf)],
                         axis=1)
    h = jax.lax.dot_general(mf, w1t_ref[...].astype(bf),
                            (((1,), (0,)), ((), ())),
                            preferred_element_type=jnp.float32) + b1_ref[...]
    h = 0.5 * h * (1.0 + jax.lax.erf(h * 0.7071067811865476))
    o = jax.lax.dot_general(h.astype(bf), w2t_ref[...].astype(bf),
                            (((1,), (0,)), ((), ())),
                            preferred_element_type=jnp.float32) + b2_ref[...]
    gate = jax.nn.sigmoid(gb_ref[0, 0])
    bpe = bpe_ref[...]
    has_morph = (rid_ref[...] >= 0) & (pat >= 0)
    out_ref[...] = jnp.where(has_morph, gate * o + (1.0 - gate) * bpe, bpe)


def _tc_compute(pat, rid, root_vecs, affix_vecs, bpe_emb,
                vu, uu, w1t, b1, w2t, b2, gate_bias):
    n, dim = bpe_emb.shape
    t = _TOKEN_BLOCK
    grid = (n // t,)

    def tok_spec(shape):
        nd = len(shape)
        return pl.BlockSpec((t,) + tuple(shape[1:]),
                            lambda i, nd=nd: (i,) + (0,) * (nd - 1))

    def full_spec(arr):
        nd = arr.ndim
        return pl.BlockSpec(arr.shape, lambda i, nd=nd: (0,) * nd)

    return pl.pallas_call(
        _morph_body,
        grid=grid,
        in_specs=[
            tok_spec(pat.shape),
            tok_spec(rid.shape),
            tok_spec(root_vecs.shape),
            tok_spec(affix_vecs.shape),
            tok_spec(bpe_emb.shape),
            full_spec(vu),
            full_spec(uu),
            full_spec(w1t),
            full_spec(b1),
            full_spec(w2t),
            full_spec(b2),
            full_spec(gate_bias),
        ],
        out_specs=tok_spec((n, dim)),
        out_shape=jax.ShapeDtypeStruct((n, dim), jnp.float32),
    )(pat, rid, root_vecs, affix_vecs, bpe_emb, vu, uu, w1t, b1, w2t, b2,
      gate_bias)


def kernel(token_ids, root_ids, pattern_ids, affix_ids, root_emb, transform_U,
           transform_V, shared_U, shared_V, affix_emb, W1, b1, W2, b2,
           bpe_table, gate_bias):
    b, s = token_ids.shape
    n = b * s
    dim = bpe_table.shape[1]
    num_pat = transform_U.shape[0]
    root_dim = root_emb.shape[1]

    tok = jnp.clip(token_ids.reshape(n).astype(jnp.int32), 0, bpe_table.shape[0] - 1)
    rid_raw = root_ids.reshape(n, 1).astype(jnp.int32)
    pat_raw = pattern_ids.reshape(n, 1).astype(jnp.int32)
    rid_safe = jnp.clip(rid_raw[:, 0], 0, root_emb.shape[0] - 1)
    aid_safe = jnp.clip(affix_ids.reshape(n).astype(jnp.int32), 0, affix_emb.shape[0] - 1)

    affix128 = jnp.pad(affix_emb, ((0, 0), (0, 128 - affix_emb.shape[1])))

    bpe_emb, root_vecs, affix_vecs = _sc_gather(
        bpe_table, root_emb, affix128, tok, rid_safe, aid_safe)

    vu = jnp.concatenate(
        [jnp.transpose(transform_V, (2, 0, 1)).reshape(root_dim, num_pat * _RANK),
         shared_V.T], axis=1)
    uu = jnp.concatenate(
        [jnp.transpose(transform_U, (0, 2, 1)).reshape(num_pat * _RANK, root_dim),
         shared_U.T], axis=0)

    out = _tc_compute(pat_raw, rid_raw, root_vecs, affix_vecs, bpe_emb,
                      vu, uu, W1.T, b1.reshape(1, dim), W2.T, b2.reshape(1, dim),
                      gate_bias.reshape(1, 1))
    return out.reshape(b, s, dim)
